# baseline (speedup 1.0000x reference)
import functools

import jax
import jax.numpy as jnp
from jax import lax
from jax.experimental import pallas as pl
from jax.experimental.pallas import tpu as pltpu
from jax.experimental.pallas import tpu_sc as plsc

_VOCAB = 100000
_DIM = 64
_BATCH = 4096

_TB = 64
_NB = _BATCH // _TB
_NBUF = 2


def _sc_gather(idx, table):
    info = plsc.get_sparse_core_info()
    nc, ns = info.num_cores, info.num_subcores
    nw = nc * ns
    b_per_w = _BATCH // nw

    mesh = plsc.VectorSubcoreMesh(core_axis_name="c", subcore_axis_name="s")

    @functools.partial(
        pl.kernel,
        out_type=jax.ShapeDtypeStruct((_BATCH, _DIM), jnp.float32),
        mesh=mesh,
        scratch_types=[
            pltpu.VMEM((b_per_w,), jnp.int32),
            pltpu.VMEM((b_per_w, _DIM), jnp.float32),
            pltpu.SemaphoreType.DMA,
        ],
        compiler_params=pltpu.CompilerParams(use_tc_tiling_on_sc=False),
    )
    def gather_kernel(idx_hbm, table_hbm, out_hbm, idx_v, rows_v, sem):
        wid = lax.axis_index("s") * nc + lax.axis_index("c")
        base = wid * b_per_w
        pltpu.sync_copy(idx_hbm.at[pl.ds(base, b_per_w)], idx_v)
        pltpu.async_copy(table_hbm.at[idx_v], rows_v, sem).wait()
        pltpu.sync_copy(rows_v, out_hbm.at[pl.ds(base, b_per_w)])

    return gather_kernel(idx, table)


def _decoder_body(emb_ref, wt_ref, b_ref, out_ref, buf, sems):
    g = pl.program_id(0)

    def dma_for(gg, sl):
        return pltpu.make_async_copy(
            buf.at[sl],
            out_ref.at[pl.ds(gg * _TB, _TB), :],
            sems.at[sl],
        )

    slot = lax.rem(g, _NBUF)

    @pl.when(g >= _NBUF)
    def _():
        dma_for(g - _NBUF, slot).wait()

    acc = jnp.dot(
        emb_ref[pl.ds(g * _TB, _TB), :],
        wt_ref[...],
        preferred_element_type=jnp.float32,
    )
    buf[slot] = acc + b_ref[...]
    dma_for(g, slot).start()

    @pl.when(g == _NB - 1)
    def _():
        for k in range(_NBUF):
            gg = _NB - _NBUF + k
            dma_for(gg, lax.rem(gg, _NBUF)).wait()


def _tc_decoder(emb, wt, bias):
    return pl.pallas_call(
        _decoder_body,
        grid=(_NB,),
        in_specs=[
            pl.BlockSpec((_BATCH, _DIM), lambda i: (0, 0)),
            pl.BlockSpec((_DIM, _VOCAB), lambda i: (0, 0)),
            pl.BlockSpec((1, _VOCAB), lambda i: (0, 0)),
        ],
        out_specs=pl.BlockSpec(memory_space=pl.ANY),
        out_shape=jax.ShapeDtypeStruct((_BATCH, _VOCAB), jnp.float32),
        scratch_shapes=[
            pltpu.VMEM((_NBUF, _TB, _VOCAB), jnp.float32),
            pltpu.SemaphoreType.DMA((_NBUF,)),
        ],
        compiler_params=pltpu.CompilerParams(
            dimension_semantics=("arbitrary",),
            vmem_limit_bytes=100_000_000,
            needs_layout_passes=True,
        ),
    )(emb, wt, bias)


def kernel(one_hot_central_word, embedding_table, decoder_weight, decoder_bias):
    idx = one_hot_central_word.astype(jnp.int32)
    emb = jnp.take(embedding_table, idx, axis=0)
    wt = decoder_weight.T.astype(jnp.bfloat16)
    return _tc_decoder(
        emb.astype(jnp.bfloat16), wt, decoder_bias.reshape(1, _VOCAB)
    )


# speedup vs baseline: 2.9305x; 2.9305x over previous
import functools

import jax
import jax.numpy as jnp
from jax import lax
from jax.experimental import pallas as pl
from jax.experimental.pallas import tpu as pltpu
from jax.experimental.pallas import tpu_sc as plsc

_VOCAB = 100000
_DIM = 64
_BATCH = 4096

_TVR = 1024


def _sc_gather(idx, table):
    info = plsc.get_sparse_core_info()
    nc, ns = info.num_cores, info.num_subcores
    nw = nc * ns
    b_per_w = _BATCH // nw

    mesh = plsc.VectorSubcoreMesh(core_axis_name="c", subcore_axis_name="s")

    @functools.partial(
        pl.kernel,
        out_type=jax.ShapeDtypeStruct((_BATCH, _DIM), jnp.float32),
        mesh=mesh,
        scratch_types=[
            pltpu.VMEM((b_per_w,), jnp.int32),
            pltpu.VMEM((b_per_w, _DIM), jnp.float32),
            pltpu.SemaphoreType.DMA,
        ],
        compiler_params=pltpu.CompilerParams(use_tc_tiling_on_sc=False),
    )
    def gather_kernel(idx_hbm, table_hbm, out_hbm, idx_v, rows_v, sem):
        wid = lax.axis_index("s") * nc + lax.axis_index("c")
        base = wid * b_per_w
        pltpu.sync_copy(idx_hbm.at[pl.ds(base, b_per_w)], idx_v)
        pltpu.async_copy(table_hbm.at[idx_v], rows_v, sem).wait()
        pltpu.sync_copy(rows_v, out_hbm.at[pl.ds(base, b_per_w)])

    return gather_kernel(idx, table)


def _decoder_body(w_ref, embt_ref, b_ref, out_ref):
    out_ref[...] = jnp.dot(
        w_ref[...],
        embt_ref[...],
        preferred_element_type=jnp.float32,
    ) + b_ref[...]


def _tc_decoder(embt, w, bias_col):
    return pl.pallas_call(
        _decoder_body,
        grid=(pl.cdiv(_VOCAB, _TVR),),
        in_specs=[
            pl.BlockSpec((_TVR, _DIM), lambda i: (i, 0)),
            pl.BlockSpec((_DIM, _BATCH), lambda i: (0, 0)),
            pl.BlockSpec((_TVR, 1), lambda i: (i, 0)),
        ],
        out_specs=pl.BlockSpec((_TVR, _BATCH), lambda i: (i, 0)),
        out_shape=jax.ShapeDtypeStruct((_VOCAB, _BATCH), jnp.float32),
        compiler_params=pltpu.CompilerParams(
            dimension_semantics=("arbitrary",),
            vmem_limit_bytes=100_000_000,
        ),
    )(w, embt, bias_col)


def kernel(one_hot_central_word, embedding_table, decoder_weight, decoder_bias):
    idx = one_hot_central_word.astype(jnp.int32)
    emb = _sc_gather(idx, embedding_table)
    embt = emb.T.astype(jnp.bfloat16)
    w = decoder_weight.astype(jnp.bfloat16)
    out_t = _tc_decoder(embt, w, decoder_bias.reshape(_VOCAB, 1))
    return out_t.T


# speedup vs baseline: 2.9956x; 1.0222x over previous
import functools

import jax
import jax.numpy as jnp
from jax import lax
from jax.experimental import pallas as pl
from jax.experimental.pallas import tpu as pltpu
from jax.experimental.pallas import tpu_sc as plsc

_VOCAB = 100000
_DIM = 64
_BATCH = 4096

_TVR = 1024
_KAUG = 80
_TPAD = 128


def _sc_gather(idx, table):
    info = plsc.get_sparse_core_info()
    nc, ns = info.num_cores, info.num_subcores
    nw = nc * ns
    b_per_w = _BATCH // nw

    mesh = plsc.VectorSubcoreMesh(core_axis_name="c", subcore_axis_name="s")

    @functools.partial(
        pl.kernel,
        out_type=jax.ShapeDtypeStruct((_BATCH, _TPAD), jnp.float32),
        mesh=mesh,
        scratch_types=[
            pltpu.VMEM((b_per_w,), jnp.int32),
            pltpu.VMEM((b_per_w, _TPAD), jnp.float32),
            pltpu.SemaphoreType.DMA,
        ],
        compiler_params=pltpu.CompilerParams(use_tc_tiling_on_sc=False),
    )
    def gather_kernel(idx_hbm, table_hbm, out_hbm, idx_v, rows_v, sem):
        wid = lax.axis_index("s") * nc + lax.axis_index("c")
        base = wid * b_per_w
        pltpu.sync_copy(idx_hbm.at[pl.ds(base, b_per_w)], idx_v)
        pltpu.async_copy(table_hbm.at[idx_v], rows_v, sem).wait()
        pltpu.sync_copy(rows_v, out_hbm.at[pl.ds(base, b_per_w)])

    return gather_kernel(idx, table)


def _decoder_body(w_ref, embt_ref, out_ref):
    out_ref[...] = jnp.dot(
        w_ref[...],
        embt_ref[...],
        preferred_element_type=jnp.float32,
    )


def _tc_decoder(embt, w):
    return pl.pallas_call(
        _decoder_body,
        grid=(pl.cdiv(_VOCAB, _TVR),),
        in_specs=[
            pl.BlockSpec((_TVR, _KAUG), lambda i: (i, 0)),
            pl.BlockSpec((_KAUG, _BATCH), lambda i: (0, 0)),
        ],
        out_specs=pl.BlockSpec((_TVR, _BATCH), lambda i: (i, 0)),
        out_shape=jax.ShapeDtypeStruct((_VOCAB, _BATCH), jnp.float32),
        compiler_params=pltpu.CompilerParams(
            dimension_semantics=("arbitrary",),
            vmem_limit_bytes=100_000_000,
        ),
    )(w, embt)


def kernel(one_hot_central_word, embedding_table, decoder_weight, decoder_bias):
    idx = one_hot_central_word.astype(jnp.int32)
    table128 = jnp.pad(embedding_table, ((0, 0), (0, _TPAD - _DIM)))
    emb = _sc_gather(idx, table128)[:, :_DIM]
    embt = jnp.concatenate(
        [
            emb.T.astype(jnp.bfloat16),
            jnp.ones((1, _BATCH), jnp.bfloat16),
            jnp.zeros((_KAUG - _DIM - 1, _BATCH), jnp.bfloat16),
        ],
        axis=0,
    )
    w = jnp.concatenate(
        [
            decoder_weight.astype(jnp.bfloat16),
            decoder_bias.astype(jnp.bfloat16)[:, None],
            jnp.zeros((_VOCAB, _KAUG - _DIM - 1), jnp.bfloat16),
        ],
        axis=1,
    )
    out_t = _tc_decoder(embt, w)
    return out_t.T


# speedup vs baseline: 3.0041x; 1.0028x over previous
import functools

import jax
import jax.numpy as jnp
from jax import lax
from jax.experimental import pallas as pl
from jax.experimental.pallas import tpu as pltpu
from jax.experimental.pallas import tpu_sc as plsc

_VOCAB = 100000
_DIM = 64
_BATCH = 4096

_TVR = 1024
_KAUG = 65
_TPAD = 128


def _sc_gather(idx, table):
    info = plsc.get_sparse_core_info()
    nc, ns = info.num_cores, info.num_subcores
    nw = nc * ns
    b_per_w = _BATCH // nw

    mesh = plsc.VectorSubcoreMesh(core_axis_name="c", subcore_axis_name="s")

    @functools.partial(
        pl.kernel,
        out_type=jax.ShapeDtypeStruct((_BATCH, _TPAD), jnp.float32),
        mesh=mesh,
        scratch_types=[
            pltpu.VMEM((b_per_w,), jnp.int32),
            pltpu.VMEM((b_per_w, _TPAD), jnp.float32),
            pltpu.SemaphoreType.DMA,
        ],
        compiler_params=pltpu.CompilerParams(use_tc_tiling_on_sc=False),
    )
    def gather_kernel(idx_hbm, table_hbm, out_hbm, idx_v, rows_v, sem):
        wid = lax.axis_index("s") * nc + lax.axis_index("c")
        base = wid * b_per_w
        pltpu.sync_copy(idx_hbm.at[pl.ds(base, b_per_w)], idx_v)
        pltpu.async_copy(table_hbm.at[idx_v], rows_v, sem).wait()
        pltpu.sync_copy(rows_v, out_hbm.at[pl.ds(base, b_per_w)])

    return gather_kernel(idx, table)


def _decoder_body(w_ref, embt_ref, out_ref):
    out_ref[...] = jnp.dot(
        w_ref[...],
        embt_ref[...],
        preferred_element_type=jnp.float32,
    )


def _tc_decoder(embt, w):
    return pl.pallas_call(
        _decoder_body,
        grid=(pl.cdiv(_VOCAB, _TVR),),
        in_specs=[
            pl.BlockSpec((_TVR, _KAUG), lambda i: (i, 0)),
            pl.BlockSpec((_KAUG, _BATCH), lambda i: (0, 0)),
        ],
        out_specs=pl.BlockSpec((_TVR, _BATCH), lambda i: (i, 0)),
        out_shape=jax.ShapeDtypeStruct((_VOCAB, _BATCH), jnp.float32),
        compiler_params=pltpu.CompilerParams(
            dimension_semantics=("arbitrary",),
            vmem_limit_bytes=100_000_000,
        ),
    )(w, embt)


def kernel(one_hot_central_word, embedding_table, decoder_weight, decoder_bias):
    idx = one_hot_central_word.astype(jnp.int32)
    table128 = jnp.pad(embedding_table, ((0, 0), (0, _TPAD - _DIM)))
    emb = _sc_gather(idx, table128)[:, :_DIM]
    embt = jnp.concatenate(
        [
            emb.T.astype(jnp.bfloat16),
            jnp.ones((1, _BATCH), jnp.bfloat16),
        ],
        axis=0,
    )
    w = jnp.concatenate(
        [
            decoder_weight.astype(jnp.bfloat16),
            decoder_bias.astype(jnp.bfloat16)[:, None],
        ],
        axis=1,
    )
    out_t = _tc_decoder(embt, w)
    return out_t.T
